# baseline (speedup 1.0000x reference)
import functools

import jax
import jax.numpy as jnp
from jax import lax
from jax.experimental import pallas as pl
from jax.experimental.pallas import tpu as pltpu
from jax.experimental.pallas import tpu_sc as plsc

L = 3
_NC = 2
_NS = 16
_NW = _NC * _NS


def _acc_rows(N):
    return -(-N // (_NS * 8)) * _NS * 8


def _sc_plan(N, E, chunk=128):
    ew = E // _NW
    nch_s = -(-ew // chunk)
    if nch_s % 2:
        nch_s += 1
    ech = nch_s + 4
    return ew, nch_s, ech


def _make_sc_agg(N, D, E, chunk=128):
    ew, nch_s, ech = _sc_plan(N, E, chunk)
    acc_n = _acc_rows(N)
    rpt = acc_n // _NS
    assert ew * _NW == E and chunk <= 128 and chunk % 8 == 0
    mesh = plsc.VectorSubcoreMesh(core_axis_name="c", subcore_axis_name="s")

    @functools.partial(
        pl.kernel,
        mesh=mesh,
        out_type=jax.ShapeDtypeStruct((_NC, acc_n, D), jnp.float32),
        scratch_types=[
            pltpu.VMEM((chunk,), jnp.int32),
            pltpu.VMEM((chunk,), jnp.int32),
            pltpu.VMEM((chunk,), jnp.int32),
            pltpu.VMEM((chunk,), jnp.int32),
            pltpu.VMEM((chunk, D), jnp.float32),
            pltpu.VMEM((chunk, D), jnp.float32),
            pltpu.VMEM_SHARED((acc_n, D), jnp.float32),
            pltpu.SemaphoreType.DMA,
            pltpu.SemaphoreType.DMA,
            pltpu.SemaphoreType.DMA,
            pltpu.SemaphoreType.DMA,
            pltpu.SemaphoreType.DMA,
            pltpu.SemaphoreType.DMA,
        ],
    )
    def sc_agg(r_hbm, src_hbm, dst_hbm, zero_hbm, out_hbm,
               si0, si1, di0, di1, buf0, buf1, acc_sh,
               sg0, sg1, ss0, ss1, sx0, sx1):
        c = lax.axis_index("c")
        s = lax.axis_index("s")
        wid = s * _NC + c
        si = (si0, si1)
        di = (di0, di1)
        bufs = (buf0, buf1)
        sg = (sg0, sg1)
        ss = (ss0, ss1)
        sx = (sx0, sx1)

        def idx_off(j):
            return (j * _NW + wid) * chunk

        def load_si(j, b):
            pltpu.async_copy(src_hbm.at[pl.ds(idx_off(j), chunk)],
                             si[b], sx[b])

        def load_di(j, b):
            pltpu.async_copy(dst_hbm.at[pl.ds(idx_off(j), chunk)],
                             di[b], ss[b])

        def wait_si(b):
            pltpu.make_async_copy(src_hbm.at[pl.ds(0, chunk)],
                                  si[b], sx[b]).wait()

        def wait_di(b):
            pltpu.make_async_copy(dst_hbm.at[pl.ds(0, chunk)],
                                  di[b], ss[b]).wait()

        load_si(0, 0)
        load_si(1, 1)
        load_di(0, 0)
        load_di(1, 1)

        pltpu.sync_copy(zero_hbm, acc_sh.at[pl.ds(s * rpt, rpt)])
        plsc.subcore_barrier()

        def body(i, carry):
            for b in range(2):
                j = 2 * i + b
                wait_si(b)
                wait_di(b)
                pltpu.async_copy(r_hbm.at[si[b]], bufs[b], sg[b]).wait()
                pltpu.sync_copy(bufs[b], acc_sh.at[di[b]], add=True)
                load_si(j + 2, b)
                load_di(j + 2, b)
            return carry

        lax.fori_loop(0, nch_s // 2, body, 0)
        wait_si(0)
        wait_si(1)
        wait_di(0)
        wait_di(1)

        plsc.subcore_barrier()
        pltpu.sync_copy(acc_sh.at[pl.ds(s * rpt, rpt)],
                        out_hbm.at[c, pl.ds(s * rpt, rpt)])

    return sc_agg


def _dense_body(h_ref, agg_ref, w1_ref, b1_ref, g1_ref, bt1_ref,
                w2_ref, b2_ref, g2_ref, bt2_ref, eps_ref, o_ref, *, last):
    n = h_ref.shape[0]
    z = (1.0 + eps_ref[0, 0]) * h_ref[...] + agg_ref[0, :n] + agg_ref[1, :n]
    z = lax.dot_general(z, w1_ref[...], (((1,), (1,)), ((), ())),
                        preferred_element_type=jnp.float32) + b1_ref[...]
    mu = jnp.mean(z, axis=0, keepdims=True)
    var = jnp.mean((z - mu) ** 2, axis=0, keepdims=True)
    z = (z - mu) * lax.rsqrt(var + 1e-5) * g1_ref[...] + bt1_ref[...]
    z = jnp.maximum(z, 0.0)
    z = lax.dot_general(z, w2_ref[...], (((1,), (1,)), ((), ())),
                        preferred_element_type=jnp.float32) + b2_ref[...]
    mu = jnp.mean(z, axis=0, keepdims=True)
    var = jnp.mean((z - mu) ** 2, axis=0, keepdims=True)
    z = (z - mu) * lax.rsqrt(var + 1e-5) * g2_ref[...] + bt2_ref[...]
    if not last:
        z = jnp.maximum(z, 0.0)
    o_ref[...] = z


def _dense(h, agg, w1, b1, g1, bt1, w2, b2, g2, bt2, eps_s, last):
    return pl.pallas_call(
        functools.partial(_dense_body, last=last),
        out_shape=jax.ShapeDtypeStruct(h.shape, jnp.float32),
    )(h, agg, w1, b1.reshape(1, -1), g1.reshape(1, -1), bt1.reshape(1, -1),
      w2, b2.reshape(1, -1), g2.reshape(1, -1), bt2.reshape(1, -1),
      eps_s.reshape(1, 1))


def _relu_body(x_ref, o_ref):
    o_ref[...] = jnp.maximum(x_ref[...], 0.0)


def _relu(x):
    return pl.pallas_call(
        _relu_body, out_shape=jax.ShapeDtypeStruct(x.shape, x.dtype))(x)


def kernel(x, edge_index, W1, b1, g1, bt1, W2, b2, eps, g2, bt2):
    N, D = x.shape
    E = edge_index.shape[1]
    chunk = 128
    ew, nch_s, ech = _sc_plan(N, E, chunk)
    acc_n = _acc_rows(N)
    pad = ech * chunk - ew
    src2d = jnp.pad(edge_index[0].reshape(_NW, ew), ((0, 0), (0, pad)),
                    constant_values=0)
    scratch = acc_n - N
    dst_pad = (N + (jnp.arange(_NW, dtype=jnp.int32)[:, None] * 8
                    + jnp.arange(pad, dtype=jnp.int32)[None, :]) % scratch)
    dst2d = jnp.concatenate([edge_index[1].reshape(_NW, ew), dst_pad], axis=1)
    src1d = src2d.reshape(_NW, ech, chunk).transpose(1, 0, 2).reshape(-1)
    dst1d = dst2d.reshape(_NW, ech, chunk).transpose(1, 0, 2).reshape(-1)
    zeros = jnp.zeros((acc_n // _NS, D), jnp.float32)

    sc_agg = _make_sc_agg(N, D, E, chunk)

    h = x
    r = _relu(x)
    for l in range(L):
        agg = sc_agg(r, src1d, dst1d, zeros)
        h = _dense(h, agg, W1[l], b1[l], g1[l], bt1[l],
                   W2[l], b2[l], g2[l], bt2[l], eps[l], last=(l == L - 1))
        r = h
    return h


# speedup vs baseline: 1.6333x; 1.6333x over previous
import functools

import jax
import jax.numpy as jnp
from jax import lax
from jax.experimental import pallas as pl
from jax.experimental.pallas import tpu as pltpu
from jax.experimental.pallas import tpu_sc as plsc

L = 3
_NC = 2
_NS = 16
_NW = _NC * _NS


def _acc_rows(N):
    return -(-N // (_NS * 8)) * _NS * 8


def _sc_plan(N, E, chunk=128):
    ew = E // _NW
    nch_s = -(-ew // chunk)
    if nch_s % 2:
        nch_s += 1
    ech = nch_s + 4
    return ew, nch_s, ech


def _make_sc_agg(N, D, E, chunk=128):
    ew, nch_s, ech = _sc_plan(N, E, chunk)
    acc_n = _acc_rows(N)
    rpt = acc_n // _NS
    assert ew * _NW == E and chunk <= 128 and chunk % 8 == 0
    mesh = plsc.VectorSubcoreMesh(core_axis_name="c", subcore_axis_name="s")

    @functools.partial(
        pl.kernel,
        mesh=mesh,
        out_type=jax.ShapeDtypeStruct((_NC, acc_n, D), jnp.float32),
        scratch_types=[
            pltpu.VMEM((chunk,), jnp.int32),
            pltpu.VMEM((chunk,), jnp.int32),
            pltpu.VMEM((chunk,), jnp.int32),
            pltpu.VMEM((chunk,), jnp.int32),
            pltpu.VMEM((chunk, D), jnp.float32),
            pltpu.VMEM((chunk, D), jnp.float32),
            pltpu.VMEM_SHARED((acc_n, D), jnp.float32),
            pltpu.SemaphoreType.DMA,
            pltpu.SemaphoreType.DMA,
            pltpu.SemaphoreType.DMA,
            pltpu.SemaphoreType.DMA,
            pltpu.SemaphoreType.DMA,
            pltpu.SemaphoreType.DMA,
        ],
    )
    def sc_agg(r_hbm, src_hbm, dst_hbm, zero_hbm, out_hbm,
               si0, si1, di0, di1, buf0, buf1, acc_sh,
               sg0, sg1, ss0, ss1, sx0, sx1):
        c = lax.axis_index("c")
        s = lax.axis_index("s")
        wid = s * _NC + c
        si = (si0, si1)
        di = (di0, di1)
        bufs = (buf0, buf1)
        sg = (sg0, sg1)
        ss = (ss0, ss1)
        sx = (sx0, sx1)

        def idx_off(j):
            return (j * _NW + wid) * chunk

        def load_si(j, b):
            pltpu.async_copy(src_hbm.at[pl.ds(idx_off(j), chunk)],
                             si[b], sx[b])

        def load_di(j, b):
            pltpu.async_copy(dst_hbm.at[pl.ds(idx_off(j), chunk)],
                             di[b], ss[b])

        def wait_si(b):
            pltpu.make_async_copy(src_hbm.at[pl.ds(0, chunk)],
                                  si[b], sx[b]).wait()

        def wait_di(b):
            pltpu.make_async_copy(dst_hbm.at[pl.ds(0, chunk)],
                                  di[b], ss[b]).wait()

        load_si(0, 0)
        load_si(1, 1)
        load_di(0, 0)
        load_di(1, 1)

        pltpu.sync_copy(zero_hbm, acc_sh.at[pl.ds(s * rpt, rpt)])
        plsc.subcore_barrier()

        def body(i, carry):
            for b in range(2):
                j = 2 * i + b
                wait_si(b)
                wait_di(b)
                pltpu.async_copy(r_hbm.at[si[b]], bufs[b], sg[b]).wait()
                pltpu.sync_copy(bufs[b], acc_sh.at[di[b]], add=True)
                load_si(j + 2, b)
                load_di(j + 2, b)
            return carry

        lax.fori_loop(0, nch_s // 2, body, 0)
        wait_si(0)
        wait_si(1)
        wait_di(0)
        wait_di(1)

        plsc.subcore_barrier()
        pltpu.sync_copy(acc_sh.at[pl.ds(s * rpt, rpt)],
                        out_hbm.at[c, pl.ds(s * rpt, rpt)])

    return sc_agg


def _dense_body(h_ref, agg_ref, w1_ref, b1_ref, g1_ref, bt1_ref,
                w2_ref, b2_ref, g2_ref, bt2_ref, eps_ref, o_ref, *, last):
    n = h_ref.shape[0]
    z = (1.0 + eps_ref[0, 0]) * h_ref[...] + agg_ref[0, :n] + agg_ref[1, :n]
    z = lax.dot_general(z, w1_ref[...], (((1,), (1,)), ((), ())),
                        preferred_element_type=jnp.float32) + b1_ref[...]
    mu = jnp.mean(z, axis=0, keepdims=True)
    var = jnp.mean((z - mu) ** 2, axis=0, keepdims=True)
    z = (z - mu) * lax.rsqrt(var + 1e-5) * g1_ref[...] + bt1_ref[...]
    z = jnp.maximum(z, 0.0)
    z = lax.dot_general(z, w2_ref[...], (((1,), (1,)), ((), ())),
                        preferred_element_type=jnp.float32) + b2_ref[...]
    mu = jnp.mean(z, axis=0, keepdims=True)
    var = jnp.mean((z - mu) ** 2, axis=0, keepdims=True)
    z = (z - mu) * lax.rsqrt(var + 1e-5) * g2_ref[...] + bt2_ref[...]
    if not last:
        z = jnp.maximum(z, 0.0)
    o_ref[...] = z


def _dense(h, agg, w1, b1, g1, bt1, w2, b2, g2, bt2, eps_s, last):
    return pl.pallas_call(
        functools.partial(_dense_body, last=last),
        out_shape=jax.ShapeDtypeStruct(h.shape, jnp.float32),
    )(h, agg, w1, b1.reshape(1, -1), g1.reshape(1, -1), bt1.reshape(1, -1),
      w2, b2.reshape(1, -1), g2.reshape(1, -1), bt2.reshape(1, -1),
      eps_s.reshape(1, 1))


def _relu_body(x_ref, o_ref):
    o_ref[...] = jnp.maximum(x_ref[...], 0.0)


def _relu(x):
    return pl.pallas_call(
        _relu_body, out_shape=jax.ShapeDtypeStruct(x.shape, x.dtype))(x)


def kernel(x, edge_index, W1, b1, g1, bt1, W2, b2, eps, g2, bt2):
    N, D = x.shape
    E = edge_index.shape[1]
    chunk = 120
    ew, nch_s, ech = _sc_plan(N, E, chunk)
    acc_n = _acc_rows(N)
    pad = ech * chunk - ew
    src2d = jnp.pad(edge_index[0].reshape(_NW, ew), ((0, 0), (0, pad)),
                    constant_values=0)
    scratch = acc_n - N
    dst_pad = (N + (jnp.arange(_NW, dtype=jnp.int32)[:, None] * 8
                    + jnp.arange(pad, dtype=jnp.int32)[None, :]) % scratch)
    dst2d = jnp.concatenate([edge_index[1].reshape(_NW, ew), dst_pad], axis=1)
    src1d = src2d.reshape(_NW, ech, chunk).transpose(1, 0, 2).reshape(-1)
    dst1d = dst2d.reshape(_NW, ech, chunk).transpose(1, 0, 2).reshape(-1)
    zeros = jnp.zeros((acc_n // _NS, D), jnp.float32)

    sc_agg = _make_sc_agg(N, D, E, chunk)

    h = x
    r = _relu(x)
    for l in range(L):
        agg = sc_agg(r, src1d, dst1d, zeros)
        h = _dense(h, agg, W1[l], b1[l], g1[l], bt1[l],
                   W2[l], b2[l], g2[l], bt2[l], eps[l], last=(l == L - 1))
        r = h
    return h


# speedup vs baseline: 1.6787x; 1.0278x over previous
import functools

import jax
import jax.numpy as jnp
from jax import lax
from jax.experimental import pallas as pl
from jax.experimental.pallas import tpu as pltpu
from jax.experimental.pallas import tpu_sc as plsc

L = 3
_NC = 2
_NS = 16
_NW = _NC * _NS


def _acc_rows(N):
    return -(-N // (_NS * 8)) * _NS * 8


def _sc_plan(N, E, chunk=128):
    ew = E // _NW
    nch_s = -(-ew // chunk)
    if nch_s % 2:
        nch_s += 1
    ech = nch_s + 4
    return ew, nch_s, ech


def _make_sc_agg(N, D, E, chunk=128):
    ew, nch_s, ech = _sc_plan(N, E, chunk)
    acc_n = _acc_rows(N)
    rpt = acc_n // _NS
    assert ew * _NW == E and chunk <= 128 and chunk % 8 == 0
    mesh = plsc.VectorSubcoreMesh(core_axis_name="c", subcore_axis_name="s")

    @functools.partial(
        pl.kernel,
        mesh=mesh,
        out_type=jax.ShapeDtypeStruct((_NC, acc_n, D), jnp.float32),
        scratch_types=[
            pltpu.VMEM((chunk,), jnp.int32),
            pltpu.VMEM((chunk,), jnp.int32),
            pltpu.VMEM((chunk,), jnp.int32),
            pltpu.VMEM((chunk,), jnp.int32),
            pltpu.VMEM((chunk, D), jnp.float32),
            pltpu.VMEM((chunk, D), jnp.float32),
            pltpu.VMEM_SHARED((acc_n, D), jnp.float32),
            pltpu.SemaphoreType.DMA,
            pltpu.SemaphoreType.DMA,
            pltpu.SemaphoreType.DMA,
            pltpu.SemaphoreType.DMA,
            pltpu.SemaphoreType.DMA,
            pltpu.SemaphoreType.DMA,
            pltpu.SemaphoreType.DMA,
            pltpu.SemaphoreType.DMA,
        ],
    )
    def sc_agg(r_hbm, src_hbm, dst_hbm, zero_hbm, out_hbm,
               si0, si1, di0, di1, buf0, buf1, acc_sh,
               sg0, sg1, ss0, ss1, sx0, sx1, sc0, sc1):
        c = lax.axis_index("c")
        s = lax.axis_index("s")
        wid = s * _NC + c
        si = (si0, si1)
        di = (di0, di1)
        bufs = (buf0, buf1)
        sg = (sg0, sg1)
        ss = (ss0, ss1)
        sx = (sx0, sx1)

        def idx_off(j):
            return (j * _NW + wid) * chunk

        def load_si(j, b):
            pltpu.async_copy(src_hbm.at[pl.ds(idx_off(j), chunk)],
                             si[b], sx[b])

        def load_di(j, b):
            pltpu.async_copy(dst_hbm.at[pl.ds(idx_off(j), chunk)],
                             di[b], ss[b])

        def wait_si(b):
            pltpu.make_async_copy(src_hbm.at[pl.ds(0, chunk)],
                                  si[b], sx[b]).wait()

        def wait_di(b):
            pltpu.make_async_copy(dst_hbm.at[pl.ds(0, chunk)],
                                  di[b], ss[b]).wait()

        load_si(0, 0)
        load_si(1, 1)
        load_di(0, 0)
        load_di(1, 1)

        pltpu.sync_copy(zero_hbm, acc_sh.at[pl.ds(s * rpt, rpt)])
        plsc.subcore_barrier()

        def body(i, carry):
            j = 2 * i
            wait_si(0)
            wait_si(1)
            g0 = pltpu.async_copy(r_hbm.at[si0], buf0, sg0)
            g1 = pltpu.async_copy(r_hbm.at[si1], buf1, sg1)
            wait_di(0)
            wait_di(1)
            g0.wait()
            g1.wait()
            s0 = pltpu.async_copy(buf0, acc_sh.at[di0], sc0, add=True)
            s1 = pltpu.async_copy(buf1, acc_sh.at[di1], sc1, add=True)
            s0.wait()
            s1.wait()
            load_si(j + 2, 0)
            load_si(j + 3, 1)
            load_di(j + 2, 0)
            load_di(j + 3, 1)
            return carry

        lax.fori_loop(0, nch_s // 2, body, 0)
        wait_si(0)
        wait_si(1)
        wait_di(0)
        wait_di(1)

        plsc.subcore_barrier()
        pltpu.sync_copy(acc_sh.at[pl.ds(s * rpt, rpt)],
                        out_hbm.at[c, pl.ds(s * rpt, rpt)])

    return sc_agg


def _dense_body(h_ref, agg_ref, w1_ref, b1_ref, g1_ref, bt1_ref,
                w2_ref, b2_ref, g2_ref, bt2_ref, eps_ref, o_ref, *, last):
    n = h_ref.shape[0]
    z = (1.0 + eps_ref[0, 0]) * h_ref[...] + agg_ref[0, :n] + agg_ref[1, :n]
    z = lax.dot_general(z, w1_ref[...], (((1,), (1,)), ((), ())),
                        preferred_element_type=jnp.float32) + b1_ref[...]
    mu = jnp.mean(z, axis=0, keepdims=True)
    var = jnp.mean((z - mu) ** 2, axis=0, keepdims=True)
    z = (z - mu) * lax.rsqrt(var + 1e-5) * g1_ref[...] + bt1_ref[...]
    z = jnp.maximum(z, 0.0)
    z = lax.dot_general(z, w2_ref[...], (((1,), (1,)), ((), ())),
                        preferred_element_type=jnp.float32) + b2_ref[...]
    mu = jnp.mean(z, axis=0, keepdims=True)
    var = jnp.mean((z - mu) ** 2, axis=0, keepdims=True)
    z = (z - mu) * lax.rsqrt(var + 1e-5) * g2_ref[...] + bt2_ref[...]
    if not last:
        z = jnp.maximum(z, 0.0)
    o_ref[...] = z


def _dense(h, agg, w1, b1, g1, bt1, w2, b2, g2, bt2, eps_s, last):
    return pl.pallas_call(
        functools.partial(_dense_body, last=last),
        out_shape=jax.ShapeDtypeStruct(h.shape, jnp.float32),
    )(h, agg, w1, b1.reshape(1, -1), g1.reshape(1, -1), bt1.reshape(1, -1),
      w2, b2.reshape(1, -1), g2.reshape(1, -1), bt2.reshape(1, -1),
      eps_s.reshape(1, 1))


def _relu_body(x_ref, o_ref):
    o_ref[...] = jnp.maximum(x_ref[...], 0.0)


def _relu(x):
    return pl.pallas_call(
        _relu_body, out_shape=jax.ShapeDtypeStruct(x.shape, x.dtype))(x)


def kernel(x, edge_index, W1, b1, g1, bt1, W2, b2, eps, g2, bt2):
    N, D = x.shape
    E = edge_index.shape[1]
    chunk = 120
    ew, nch_s, ech = _sc_plan(N, E, chunk)
    acc_n = _acc_rows(N)
    pad = ech * chunk - ew
    src2d = jnp.pad(edge_index[0].reshape(_NW, ew), ((0, 0), (0, pad)),
                    constant_values=0)
    scratch = acc_n - N
    dst_pad = (N + (jnp.arange(_NW, dtype=jnp.int32)[:, None] * 8
                    + jnp.arange(pad, dtype=jnp.int32)[None, :]) % scratch)
    dst2d = jnp.concatenate([edge_index[1].reshape(_NW, ew), dst_pad], axis=1)
    src1d = src2d.reshape(_NW, ech, chunk).transpose(1, 0, 2).reshape(-1)
    dst1d = dst2d.reshape(_NW, ech, chunk).transpose(1, 0, 2).reshape(-1)
    zeros = jnp.zeros((acc_n // _NS, D), jnp.float32)

    sc_agg = _make_sc_agg(N, D, E, chunk)

    h = x
    r = _relu(x)
    for l in range(L):
        agg = sc_agg(r, src1d, dst1d, zeros)
        h = _dense(h, agg, W1[l], b1[l], g1[l], bt1[l],
                   W2[l], b2[l], g2[l], bt2[l], eps[l], last=(l == L - 1))
        r = h
    return h


# speedup vs baseline: 1.7483x; 1.0414x over previous
import functools

import jax
import jax.numpy as jnp
from jax import lax
from jax.experimental import pallas as pl
from jax.experimental.pallas import tpu as pltpu
from jax.experimental.pallas import tpu_sc as plsc

L = 3
_NC = 2
_NS = 16
_NW = _NC * _NS


def _acc_rows(N):
    return -(-N // (_NS * 8)) * _NS * 8


_NB = 3


def _sc_plan(N, E, chunk=128):
    ew = E // _NW
    nch_s = -(-ew // chunk)
    nch_s = -(-nch_s // _NB) * _NB
    ech = nch_s + 2 * _NB
    return ew, nch_s, ech


def _make_sc_agg(N, D, E, chunk=128):
    ew, nch_s, ech = _sc_plan(N, E, chunk)
    acc_n = _acc_rows(N)
    rpt = acc_n // _NS
    assert ew * _NW == E and chunk <= 128 and chunk % 8 == 0
    mesh = plsc.VectorSubcoreMesh(core_axis_name="c", subcore_axis_name="s")

    @functools.partial(
        pl.kernel,
        mesh=mesh,
        out_type=jax.ShapeDtypeStruct((_NC, acc_n, D), jnp.float32),
        scratch_types=(
            [pltpu.VMEM((chunk,), jnp.int32) for _ in range(_NB)]
            + [pltpu.VMEM((chunk,), jnp.int32) for _ in range(_NB)]
            + [pltpu.VMEM((chunk, D), jnp.float32) for _ in range(_NB)]
            + [pltpu.VMEM_SHARED((acc_n, D), jnp.float32)]
            + [pltpu.SemaphoreType.DMA for _ in range(4 * _NB)]
        ),
    )
    def sc_agg(r_hbm, src_hbm, dst_hbm, zero_hbm, out_hbm, *refs):
        si = refs[:_NB]
        di = refs[_NB:2 * _NB]
        bufs = refs[2 * _NB:3 * _NB]
        acc_sh = refs[3 * _NB]
        sems = refs[3 * _NB + 1:]
        sg = sems[:_NB]
        sc = sems[_NB:2 * _NB]
        sx = sems[2 * _NB:3 * _NB]
        ss = sems[3 * _NB:4 * _NB]
        c = lax.axis_index("c")
        s = lax.axis_index("s")
        wid = s * _NC + c

        def idx_off(j):
            return (j * _NW + wid) * chunk

        def load_idx(j, b):
            pltpu.async_copy(src_hbm.at[pl.ds(idx_off(j), chunk)],
                             si[b], sx[b])
            pltpu.async_copy(dst_hbm.at[pl.ds(idx_off(j), chunk)],
                             di[b], ss[b])

        def wait_si(b):
            pltpu.make_async_copy(src_hbm.at[pl.ds(0, chunk)],
                                  si[b], sx[b]).wait()

        def wait_di(b):
            pltpu.make_async_copy(dst_hbm.at[pl.ds(0, chunk)],
                                  di[b], ss[b]).wait()

        for b in range(_NB):
            load_idx(b, b)

        pltpu.sync_copy(zero_hbm, acc_sh.at[pl.ds(s * rpt, rpt)])
        plsc.subcore_barrier()

        def body(i, carry):
            j = _NB * i
            for b in range(_NB):
                wait_si(b)
            gs = [pltpu.async_copy(r_hbm.at[si[b]], bufs[b], sg[b])
                  for b in range(_NB)]
            for b in range(_NB):
                wait_di(b)
                gs[b].wait()
            scs = [pltpu.async_copy(bufs[b], acc_sh.at[di[b]], sc[b], add=True)
                   for b in range(_NB)]
            for b in range(_NB):
                scs[b].wait()
                load_idx(j + _NB + b, b)
            return carry

        lax.fori_loop(0, nch_s // _NB, body, 0)
        for b in range(_NB):
            wait_si(b)
            wait_di(b)

        plsc.subcore_barrier()
        pltpu.sync_copy(acc_sh.at[pl.ds(s * rpt, rpt)],
                        out_hbm.at[c, pl.ds(s * rpt, rpt)])

    return sc_agg


def _dense_body(h_ref, agg_ref, w1_ref, b1_ref, g1_ref, bt1_ref,
                w2_ref, b2_ref, g2_ref, bt2_ref, eps_ref, o_ref, *, last):
    n = h_ref.shape[0]
    z = (1.0 + eps_ref[0, 0]) * h_ref[...] + agg_ref[0, :n] + agg_ref[1, :n]
    z = lax.dot_general(z, w1_ref[...], (((1,), (1,)), ((), ())),
                        preferred_element_type=jnp.float32) + b1_ref[...]
    mu = jnp.mean(z, axis=0, keepdims=True)
    var = jnp.mean((z - mu) ** 2, axis=0, keepdims=True)
    z = (z - mu) * lax.rsqrt(var + 1e-5) * g1_ref[...] + bt1_ref[...]
    z = jnp.maximum(z, 0.0)
    z = lax.dot_general(z, w2_ref[...], (((1,), (1,)), ((), ())),
                        preferred_element_type=jnp.float32) + b2_ref[...]
    mu = jnp.mean(z, axis=0, keepdims=True)
    var = jnp.mean((z - mu) ** 2, axis=0, keepdims=True)
    z = (z - mu) * lax.rsqrt(var + 1e-5) * g2_ref[...] + bt2_ref[...]
    if not last:
        z = jnp.maximum(z, 0.0)
    o_ref[...] = z


def _dense(h, agg, w1, b1, g1, bt1, w2, b2, g2, bt2, eps_s, last):
    return pl.pallas_call(
        functools.partial(_dense_body, last=last),
        out_shape=jax.ShapeDtypeStruct(h.shape, jnp.float32),
    )(h, agg, w1, b1.reshape(1, -1), g1.reshape(1, -1), bt1.reshape(1, -1),
      w2, b2.reshape(1, -1), g2.reshape(1, -1), bt2.reshape(1, -1),
      eps_s.reshape(1, 1))


def _relu_body(x_ref, o_ref):
    o_ref[...] = jnp.maximum(x_ref[...], 0.0)


def _relu(x):
    return pl.pallas_call(
        _relu_body, out_shape=jax.ShapeDtypeStruct(x.shape, x.dtype))(x)


def kernel(x, edge_index, W1, b1, g1, bt1, W2, b2, eps, g2, bt2):
    N, D = x.shape
    E = edge_index.shape[1]
    chunk = 120
    ew, nch_s, ech = _sc_plan(N, E, chunk)
    acc_n = _acc_rows(N)
    pad = ech * chunk - ew
    src2d = jnp.pad(edge_index[0].reshape(_NW, ew), ((0, 0), (0, pad)),
                    constant_values=0)
    scratch = acc_n - N
    dst_pad = (N + (jnp.arange(_NW, dtype=jnp.int32)[:, None] * 8
                    + jnp.arange(pad, dtype=jnp.int32)[None, :]) % scratch)
    dst2d = jnp.concatenate([edge_index[1].reshape(_NW, ew), dst_pad], axis=1)
    src1d = src2d.reshape(_NW, ech, chunk).transpose(1, 0, 2).reshape(-1)
    dst1d = dst2d.reshape(_NW, ech, chunk).transpose(1, 0, 2).reshape(-1)
    zeros = jnp.zeros((acc_n // _NS, D), jnp.float32)

    sc_agg = _make_sc_agg(N, D, E, chunk)

    h = x
    r = _relu(x)
    for l in range(L):
        agg = sc_agg(r, src1d, dst1d, zeros)
        h = _dense(h, agg, W1[l], b1[l], g1[l], bt1[l],
                   W2[l], b2[l], g2[l], bt2[l], eps[l], last=(l == L - 1))
        r = h
    return h
